# baseline (speedup 1.0000x reference)
import functools

import jax
import jax.numpy as jnp
from jax import lax
from jax.experimental import pallas as pl
from jax.experimental.pallas import tpu as pltpu

K_TOP = 32


def _enc_step(x, w_ref, b_ref, z_ref, store_ref, search_ref, *, cb,
              c_blk, k_top):
    bits = search_ref[...]
    t = jnp.zeros((bits.shape[0], 1), jnp.int32)
    for it in range(1, 32):
        cand = t | jnp.int32(1 << (31 - it))
        cnt = jnp.sum(jnp.where(bits >= cand, 1.0, 0.0),
                      axis=1, keepdims=True)
        t = jnp.where(cnt >= k_top, cand, t)
        j = it - 1
        if j < cb:
            sl = pl.ds(j * c_blk, c_blk)
            pre = jnp.dot(x, w_ref[:, sl], preferred_element_type=jnp.float32)
            r = jnp.maximum(pre + b_ref[:, sl], 0.0)
            store_ref[:, sl] = lax.bitcast_convert_type(r, jnp.int32)
    rfull = lax.bitcast_convert_type(bits, jnp.float32)
    z_ref[...] = jnp.where(bits >= t, rfull, 0.0)


def _enc_kernel(x_ref, w_ref, b_ref, z_ref, bits_a, bits_b, *, cb,
                c_blk, k_top):
    i = pl.program_id(0)
    x = x_ref[...]
    step = functools.partial(_enc_step, x, w_ref, b_ref, z_ref,
                             cb=cb, c_blk=c_blk, k_top=k_top)

    @pl.when(i % 2 == 0)
    def _():
        step(bits_a, bits_b)

    @pl.when(i % 2 == 1)
    def _():
        step(bits_b, bits_a)


def _encode_call(xa, w_enc, b_enc, *, r_blk, c_blk, interpret=False):
    rows, d_in = xa.shape
    d_sae = w_enc.shape[1]
    rb, cb = rows // r_blk, d_sae // c_blk
    return pl.pallas_call(
        functools.partial(_enc_kernel, cb=cb, c_blk=c_blk, k_top=K_TOP),
        grid=(rb + 1,),
        in_specs=[
            pl.BlockSpec((r_blk, d_in), lambda i: (jnp.minimum(i, rb - 1), 0)),
            pl.BlockSpec((d_in, d_sae), lambda i: (0, 0)),
            pl.BlockSpec((1, d_sae), lambda i: (0, 0)),
        ],
        out_specs=pl.BlockSpec((r_blk, d_sae),
                               lambda i: (jnp.maximum(i - 1, 0), 0)),
        out_shape=jax.ShapeDtypeStruct((rows, d_sae), jnp.float32),
        scratch_shapes=[pltpu.VMEM((r_blk, d_sae), jnp.int32),
                        pltpu.VMEM((r_blk, d_sae), jnp.int32)],
        interpret=interpret,
    )(xa, w_enc, b_enc)


def _dec_kernel(z_ref, w_ref, b_ref, x_ref, dec_ref, loss_ref, zcur_ref,
                *, kb, hb, k_blk, scale):
    i = pl.program_id(0)
    high = None
    acc = None
    for k in range(kb):
        zc = z_ref[:, pl.ds(k * k_blk, k_blk)].astype(jnp.bfloat16)
        p = jnp.dot(zc, w_ref[pl.ds(k * k_blk, k_blk), :],
                    preferred_element_type=jnp.float32)
        acc = p if acc is None else acc + p
        if k == hb - 1:
            high = acc
    b = b_ref[...]
    x = x_ref[...]
    full = acc + b
    high = high + b
    dec_ref[...] = full
    zcur_ref[...] = z_ref[...]
    ef = full - x
    eh = high - x
    s = (jnp.sum(ef * ef) + jnp.sum(eh * eh)) * scale

    @pl.when(i == 0)
    def _():
        loss_ref[0, 0] = s

    @pl.when(i > 0)
    def _():
        loss_ref[0, 0] += s


def _decode_call(z, w_dec_bf16, b_dec, xa, *, r_blk, k_blk, h, interpret=False):
    rows, d_sae = z.shape
    d_in = w_dec_bf16.shape[1]
    rb, kb = rows // r_blk, d_sae // k_blk
    hb = h // k_blk
    half = rb // 2
    scale = 1.0 / (float(rows // 2) * float(d_in))
    return pl.pallas_call(
        functools.partial(_dec_kernel, kb=kb, hb=hb, k_blk=k_blk, scale=scale),
        grid=(rb,),
        in_specs=[
            pl.BlockSpec((r_blk, d_sae), lambda i: (i, 0)),
            pl.BlockSpec((d_sae, d_in), lambda i: (0, 0)),
            pl.BlockSpec((1, d_in), lambda i: (0, 0)),
            pl.BlockSpec((r_blk, d_in), lambda i: (i, 0)),
        ],
        out_specs=[
            pl.BlockSpec((r_blk, d_in), lambda i: (jnp.maximum(i - half, 0), 0)),
            pl.BlockSpec(memory_space=pltpu.SMEM),
            pl.BlockSpec((r_blk, d_sae), lambda i: (jnp.maximum(i - half, 0), 0)),
        ],
        out_shape=[
            jax.ShapeDtypeStruct((rows // 2, d_in), jnp.float32),
            jax.ShapeDtypeStruct((1, 1), jnp.float32),
            jax.ShapeDtypeStruct((rows // 2, d_sae), jnp.float32),
        ],
        interpret=interpret,
    )(z, w_dec_bf16, b_dec, xa)


def _nce_kernel(a_ref, b_ref, out_ref, cmax, csum, acc, bh_s, *, rb, r_blk, n):
    i = pl.program_id(0)

    @pl.when(i == 0)
    def _():
        b = b_ref[...]
        nb = jnp.maximum(jnp.sqrt(jnp.sum(b * b, axis=1, keepdims=True)),
                         1e-8)
        bh_s[...] = (b / nb).astype(jnp.bfloat16)

    a = a_ref[...]
    na = jnp.maximum(jnp.sqrt(jnp.sum(a * a, axis=1, keepdims=True)), 1e-8)
    ah = a / na
    sim = lax.dot_general(ah.astype(jnp.bfloat16), bh_s[...],
                          (((1,), (1,)), ((), ())),
                          preferred_element_type=jnp.float32)
    diag = jnp.sum(ah * bh_s[pl.ds(i * r_blk, r_blk), :].astype(jnp.float32),
                   axis=1)
    mr = jnp.max(sim, axis=1)
    lse_r = jnp.log(jnp.sum(jnp.exp(sim - mr[:, None]), axis=1)) + mr
    bm = jnp.max(sim, axis=0, keepdims=True)
    s_r = jnp.sum(lse_r)
    s_d = jnp.sum(diag)

    @pl.when(i == 0)
    def _():
        cmax[...] = bm
        csum[...] = jnp.sum(jnp.exp(sim - bm), axis=0, keepdims=True)
        acc[0] = s_r
        acc[1] = s_d

    @pl.when(i > 0)
    def _():
        m_new = jnp.maximum(cmax[...], bm)
        csum[...] = (csum[...] * jnp.exp(cmax[...] - m_new)
                     + jnp.sum(jnp.exp(sim - m_new), axis=0, keepdims=True))
        cmax[...] = m_new
        acc[0] += s_r
        acc[1] += s_d

    @pl.when(i == rb - 1)
    def _():
        lse_c = jnp.log(csum[...]) + cmax[...]
        ce1 = (acc[0] - acc[1]) / n
        ce2 = (jnp.sum(lse_c) - acc[1]) / n
        out_ref[0, 0] = 0.5 * (ce1 + ce2)


def _nce_call(z, *, h, r_blk, interpret=False):
    rows = z.shape[0]
    n = rows // 2
    rb = n // r_blk
    nb = n // r_blk
    return pl.pallas_call(
        functools.partial(_nce_kernel, rb=rb, r_blk=r_blk, n=float(n)),
        grid=(rb,),
        in_specs=[
            pl.BlockSpec((r_blk, h), lambda i: (nb + i, 0)),
            pl.BlockSpec((n, h), lambda i: (0, 0)),
        ],
        out_specs=pl.BlockSpec(memory_space=pltpu.SMEM),
        out_shape=jax.ShapeDtypeStruct((1, 1), jnp.float32),
        scratch_shapes=[
            pltpu.VMEM((1, n), jnp.float32),
            pltpu.VMEM((1, n), jnp.float32),
            pltpu.SMEM((2,), jnp.float32),
            pltpu.VMEM((n, h), jnp.bfloat16),
        ],
        interpret=interpret,
    )(z, z)


def _forward(x, w_enc, w_dec, b_enc, b_dec, interpret=False):
    bsz, _, d_in = x.shape
    d_sae = w_enc.shape[1]
    h = d_sae // 2
    rows = 2 * bsz
    xa = jnp.swapaxes(x, 0, 1).reshape(rows, d_in)

    r_blk = min(128, bsz)
    c_blk = min(2048, d_sae)
    z = _encode_call(xa, w_enc, b_enc.reshape(1, -1),
                     r_blk=r_blk, c_blk=c_blk, interpret=interpret)
    k_blk = min(2048, h)
    dec_cur, l_matr, z_cur = _decode_call(z, w_dec.astype(jnp.bfloat16),
                                          b_dec.reshape(1, -1), xa,
                                          r_blk=min(256, bsz), k_blk=k_blk,
                                          h=h, interpret=interpret)
    l_contr = _nce_call(z, h=h, r_blk=min(256, bsz), interpret=interpret)
    total = l_matr[0, 0] + l_contr[0, 0]
    return total, dec_cur, z_cur


def kernel(x, W_enc, W_dec, b_enc, b_dec):
    return _forward(x, W_enc, W_dec, b_enc, b_dec)


# speedup vs baseline: 1.0015x; 1.0015x over previous
import functools

import jax
import jax.numpy as jnp
from jax import lax
from jax.experimental import pallas as pl
from jax.experimental.pallas import tpu as pltpu

K_TOP = 32


def _enc_step(x, w_ref, b_ref, z_ref, store_ref, search_ref, *, cb,
              c_blk, k_top):
    bits = search_ref[...]
    t = jnp.zeros((bits.shape[0], 1), jnp.int32)
    for it in range(1, 32):
        cand = t | jnp.int32(1 << (31 - it))
        cnt = jnp.sum(jnp.where(bits >= cand, 1.0, 0.0),
                      axis=1, keepdims=True)
        t = jnp.where(cnt >= k_top, cand, t)
        j = it - 1
        if j < cb:
            sl = pl.ds(j * c_blk, c_blk)
            pre = jnp.dot(x, w_ref[:, sl], preferred_element_type=jnp.float32)
            r = jnp.maximum(pre + b_ref[:, sl], 0.0)
            store_ref[:, sl] = lax.bitcast_convert_type(r, jnp.int32)
    rfull = lax.bitcast_convert_type(bits, jnp.float32)
    z_ref[...] = jnp.where(bits >= t, rfull, 0.0)


def _enc_kernel(x_ref, w_ref, b_ref, z_ref, bits_a, bits_b, *, cb,
                c_blk, k_top):
    i = pl.program_id(0)
    x = x_ref[...]
    step = functools.partial(_enc_step, x, w_ref, b_ref, z_ref,
                             cb=cb, c_blk=c_blk, k_top=k_top)

    @pl.when(i % 2 == 0)
    def _():
        step(bits_a, bits_b)

    @pl.when(i % 2 == 1)
    def _():
        step(bits_b, bits_a)


def _encode_call(xa, w_enc, b_enc, *, r_blk, c_blk, interpret=False):
    rows, d_in = xa.shape
    d_sae = w_enc.shape[1]
    rb, cb = rows // r_blk, d_sae // c_blk
    return pl.pallas_call(
        functools.partial(_enc_kernel, cb=cb, c_blk=c_blk, k_top=K_TOP),
        grid=(rb + 1,),
        in_specs=[
            pl.BlockSpec((r_blk, d_in), lambda i: (jnp.minimum(i, rb - 1), 0)),
            pl.BlockSpec((d_in, d_sae), lambda i: (0, 0)),
            pl.BlockSpec((1, d_sae), lambda i: (0, 0)),
        ],
        out_specs=pl.BlockSpec((r_blk, d_sae),
                               lambda i: (jnp.maximum(i - 1, 0), 0)),
        out_shape=jax.ShapeDtypeStruct((rows, d_sae), jnp.float32),
        scratch_shapes=[pltpu.VMEM((r_blk, d_sae), jnp.int32),
                        pltpu.VMEM((r_blk, d_sae), jnp.int32)],
        interpret=interpret,
    )(xa, w_enc, b_enc)


def _dec_kernel(z_ref, w_ref, b_ref, x_ref, dec_ref, loss_ref, zcur_ref,
                *, kb, hb, k_blk, scale):
    i = pl.program_id(0)
    high = None
    acc = None
    for k in range(kb):
        zc = z_ref[:, pl.ds(k * k_blk, k_blk)].astype(jnp.bfloat16)
        p = jnp.dot(zc, w_ref[pl.ds(k * k_blk, k_blk), :],
                    preferred_element_type=jnp.float32)
        acc = p if acc is None else acc + p
        if k == hb - 1:
            high = acc
    b = b_ref[...]
    x = x_ref[...]
    full = acc + b
    high = high + b
    dec_ref[...] = full
    zcur_ref[...] = z_ref[...]
    ef = full - x
    eh = high - x
    s = (jnp.sum(ef * ef) + jnp.sum(eh * eh)) * scale

    @pl.when(i == 0)
    def _():
        loss_ref[0, 0] = s

    @pl.when(i > 0)
    def _():
        loss_ref[0, 0] += s


def _decode_call(z, w_dec_bf16, b_dec, xa, *, r_blk, k_blk, h, interpret=False):
    rows, d_sae = z.shape
    d_in = w_dec_bf16.shape[1]
    rb, kb = rows // r_blk, d_sae // k_blk
    hb = h // k_blk
    half = rb // 2
    scale = 1.0 / (float(rows // 2) * float(d_in))
    return pl.pallas_call(
        functools.partial(_dec_kernel, kb=kb, hb=hb, k_blk=k_blk, scale=scale),
        grid=(rb,),
        in_specs=[
            pl.BlockSpec((r_blk, d_sae), lambda i: (i, 0)),
            pl.BlockSpec((d_sae, d_in), lambda i: (0, 0)),
            pl.BlockSpec((1, d_in), lambda i: (0, 0)),
            pl.BlockSpec((r_blk, d_in), lambda i: (i, 0)),
        ],
        out_specs=[
            pl.BlockSpec((r_blk, d_in), lambda i: (jnp.maximum(i - half, 0), 0)),
            pl.BlockSpec(memory_space=pltpu.SMEM),
            pl.BlockSpec((r_blk, d_sae), lambda i: (jnp.maximum(i - half, 0), 0)),
        ],
        out_shape=[
            jax.ShapeDtypeStruct((rows // 2, d_in), jnp.float32),
            jax.ShapeDtypeStruct((1, 1), jnp.float32),
            jax.ShapeDtypeStruct((rows // 2, d_sae), jnp.float32),
        ],
        interpret=interpret,
    )(z, w_dec_bf16, b_dec, xa)


def _nce_kernel(a_ref, b_ref, out_ref, cmax, csum, acc, bh_s, *, rb, r_blk, n):
    i = pl.program_id(0)

    @pl.when(i == 0)
    def _():
        b = b_ref[...]
        nb = jnp.maximum(jnp.sqrt(jnp.sum(b * b, axis=1, keepdims=True)),
                         1e-8)
        bh_s[...] = (b / nb).astype(jnp.bfloat16)

    a = a_ref[...]
    na = jnp.maximum(jnp.sqrt(jnp.sum(a * a, axis=1, keepdims=True)), 1e-8)
    ah = a / na
    sim = lax.dot_general(ah.astype(jnp.bfloat16), bh_s[...],
                          (((1,), (1,)), ((), ())),
                          preferred_element_type=jnp.float32)
    diag = jnp.sum(ah * bh_s[pl.ds(i * r_blk, r_blk), :].astype(jnp.float32),
                   axis=1)
    mr = jnp.max(sim, axis=1)
    lse_r = jnp.log(jnp.sum(jnp.exp(sim - mr[:, None]), axis=1)) + mr
    bm = jnp.max(sim, axis=0, keepdims=True)
    s_r = jnp.sum(lse_r)
    s_d = jnp.sum(diag)

    @pl.when(i == 0)
    def _():
        cmax[...] = bm
        csum[...] = jnp.sum(jnp.exp(sim - bm), axis=0, keepdims=True)
        acc[0] = s_r
        acc[1] = s_d

    @pl.when(i > 0)
    def _():
        m_new = jnp.maximum(cmax[...], bm)
        csum[...] = (csum[...] * jnp.exp(cmax[...] - m_new)
                     + jnp.sum(jnp.exp(sim - m_new), axis=0, keepdims=True))
        cmax[...] = m_new
        acc[0] += s_r
        acc[1] += s_d

    @pl.when(i == rb - 1)
    def _():
        lse_c = jnp.log(csum[...]) + cmax[...]
        ce1 = (acc[0] - acc[1]) / n
        ce2 = (jnp.sum(lse_c) - acc[1]) / n
        out_ref[0, 0] = 0.5 * (ce1 + ce2)


def _nce_call(z, *, h, r_blk, interpret=False):
    rows = z.shape[0]
    n = rows // 2
    rb = n // r_blk
    nb = n // r_blk
    return pl.pallas_call(
        functools.partial(_nce_kernel, rb=rb, r_blk=r_blk, n=float(n)),
        grid=(rb,),
        in_specs=[
            pl.BlockSpec((r_blk, h), lambda i: (nb + i, 0)),
            pl.BlockSpec((n, h), lambda i: (0, 0)),
        ],
        out_specs=pl.BlockSpec(memory_space=pltpu.SMEM),
        out_shape=jax.ShapeDtypeStruct((1, 1), jnp.float32),
        scratch_shapes=[
            pltpu.VMEM((1, n), jnp.float32),
            pltpu.VMEM((1, n), jnp.float32),
            pltpu.SMEM((2,), jnp.float32),
            pltpu.VMEM((n, h), jnp.bfloat16),
        ],
        interpret=interpret,
    )(z, z)


def _forward(x, w_enc, w_dec, b_enc, b_dec, interpret=False):
    bsz, _, d_in = x.shape
    d_sae = w_enc.shape[1]
    h = d_sae // 2
    rows = 2 * bsz
    xa = jnp.swapaxes(x, 0, 1).reshape(rows, d_in)

    r_blk = min(128, bsz)
    c_blk = min(1024, d_sae)
    z = _encode_call(xa, w_enc, b_enc.reshape(1, -1),
                     r_blk=r_blk, c_blk=c_blk, interpret=interpret)
    k_blk = min(2048, h)
    dec_cur, l_matr, z_cur = _decode_call(z, w_dec.astype(jnp.bfloat16),
                                          b_dec.reshape(1, -1), xa,
                                          r_blk=min(256, bsz), k_blk=k_blk,
                                          h=h, interpret=interpret)
    l_contr = _nce_call(z, h=h, r_blk=min(256, bsz), interpret=interpret)
    total = l_matr[0, 0] + l_contr[0, 0]
    return total, dec_cur, z_cur


def kernel(x, W_enc, W_dec, b_enc, b_dec):
    return _forward(x, W_enc, W_dec, b_enc, b_dec)
